# Initial kernel scaffold; baseline (speedup 1.0000x reference)
import jax
import jax.numpy as jnp
from jax.experimental import pallas as pl


def kernel(x, edge_index, adj_values, W):
    raise NotImplementedError("write your pallas kernel here")


# speedup vs baseline: 5.2960x; 5.2960x over previous
import functools

import jax
import jax.numpy as jnp
from jax import lax
from jax.experimental import pallas as pl
from jax.experimental.pallas import tpu as pltpu
from jax.experimental.pallas import tpu_sc as plsc

_N = 10000
_E = 320000
_D = 128

_B = 128
_NB_ALL = _E // _B
_NW = 32
_NB_MAIN = _NB_ALL // _NW
_NB_REM = _NB_ALL - _NB_MAIN * _NW
_ROWS_PER_TILE = 624
_TAIL_ROWS = _N - 16 * _ROWS_PER_TILE


def _matmul_body(x_ref, w_ref, o_ref):
    o_ref[...] = jnp.dot(x_ref[...], w_ref[...],
                         preferred_element_type=jnp.float32)


def _matmul(x, W):
    blk = 1000
    return pl.pallas_call(
        _matmul_body,
        grid=(_N // blk,),
        in_specs=[
            pl.BlockSpec((blk, _D), lambda i: (i, 0)),
            pl.BlockSpec((_D, _D), lambda i: (0, 0)),
        ],
        out_specs=pl.BlockSpec((blk, _D), lambda i: (i, 0)),
        out_shape=jax.ShapeDtypeStruct((_N, _D), jnp.float32),
    )(x, W)


def _combine_body(a_ref, b_ref, o_ref):
    o_ref[...] = a_ref[...] + b_ref[...]


def _combine(p0, p1):
    blk = 1000
    return pl.pallas_call(
        _combine_body,
        grid=(_N // blk,),
        in_specs=[
            pl.BlockSpec((blk, _D), lambda i: (i, 0)),
            pl.BlockSpec((blk, _D), lambda i: (i, 0)),
        ],
        out_specs=pl.BlockSpec((blk, _D), lambda i: (i, 0)),
        out_shape=jax.ShapeDtypeStruct((_N, _D), jnp.float32),
    )(p0, p1)


def _sc_aggregate(support, col2d, row2d, adj2d):
    mesh = plsc.VectorSubcoreMesh(core_axis_name="c", subcore_axis_name="s")

    @functools.partial(
        pl.kernel,
        out_type=jax.ShapeDtypeStruct((2, _N, _D), jnp.float32),
        mesh=mesh,
        scratch_types=[
            pltpu.VMEM((_B,), jnp.int32),
            pltpu.VMEM((_B,), jnp.int32),
            pltpu.VMEM((_B,), jnp.float32),
            pltpu.VMEM((_B, _D), jnp.float32),
            pltpu.VMEM_SHARED((_N, _D), jnp.float32),
            pltpu.SemaphoreType.DMA,
        ],
    )
    def agg(support_hbm, col_hbm, row_hbm, adj_hbm, out_hbm,
            col_buf, row_buf, adj_buf, rows_buf, acc, sem):
        c = lax.axis_index("c")
        s = lax.axis_index("s")
        wid = c * 16 + s

        @pl.loop(0, _B)
        def _zero(e):
            for cc in range(8):
                rows_buf[e, pl.ds(cc * 16, 16)] = jnp.zeros((16,), jnp.float32)

        base_row = s * _ROWS_PER_TILE
        for kk in range(4):
            pltpu.sync_copy(rows_buf, acc.at[pl.ds(base_row + kk * _B, _B)])
        pltpu.sync_copy(rows_buf.at[pl.ds(0, _ROWS_PER_TILE - 4 * _B)],
                        acc.at[pl.ds(base_row + 4 * _B,
                                     _ROWS_PER_TILE - 4 * _B)])

        @pl.when(s == 0)
        def _zero_tail():
            pltpu.sync_copy(rows_buf.at[pl.ds(0, _TAIL_ROWS)],
                            acc.at[pl.ds(16 * _ROWS_PER_TILE, _TAIL_ROWS)])

        plsc.subcore_barrier()

        def do_batch(r):
            pltpu.sync_copy(col_hbm.at[r], col_buf)
            pltpu.sync_copy(row_hbm.at[r], row_buf)
            pltpu.sync_copy(adj_hbm.at[r], adj_buf)
            pltpu.async_copy(support_hbm.at[col_buf], rows_buf, sem).wait()

            @pl.loop(0, _B // 16)
            def _scale(g16):
                av16 = adj_buf[pl.ds(g16 * 16, 16)]
                for i in range(16):
                    e = g16 * 16 + i
                    av = jnp.full((16,), av16[i], jnp.float32)
                    for cc in range(8):
                        sl = pl.ds(cc * 16, 16)
                        rows_buf[e, sl] = rows_buf[e, sl] * av

            pltpu.sync_copy(rows_buf, acc.at[row_buf], add=True)

        @pl.loop(0, _NB_MAIN)
        def _main(g):
            do_batch(wid * _NB_MAIN + g)

        @pl.when(wid < _NB_REM)
        def _rem():
            do_batch(_NW * _NB_MAIN + wid)

        plsc.subcore_barrier()
        pltpu.sync_copy(acc.at[pl.ds(base_row, _ROWS_PER_TILE)],
                        out_hbm.at[c, pl.ds(base_row, _ROWS_PER_TILE)])

        @pl.when(s == 0)
        def _out_tail():
            pltpu.sync_copy(acc.at[pl.ds(16 * _ROWS_PER_TILE, _TAIL_ROWS)],
                            out_hbm.at[c, pl.ds(16 * _ROWS_PER_TILE,
                                                _TAIL_ROWS)])

    return agg(support, col2d, row2d, adj2d)


def kernel(x, edge_index, adj_values, W):
    support = _matmul(x, W)
    col2d = edge_index[1].reshape(_NB_ALL, _B)
    row2d = edge_index[0].reshape(_NB_ALL, _B)
    adj2d = adj_values.reshape(_NB_ALL, _B)
    partial = _sc_aggregate(support, col2d, row2d, adj2d)
    return _combine(partial[0], partial[1])
